# Initial kernel scaffold; baseline (speedup 1.0000x reference)
import jax
import jax.numpy as jnp
from jax.experimental import pallas as pl


def kernel(losses, group_ids, group_weights):
    raise NotImplementedError("write your pallas kernel here")


# speedup vs baseline: 36.5435x; 36.5435x over previous
import functools

import jax
import jax.numpy as jnp
from jax import lax
from jax.experimental import pallas as pl
from jax.experimental.pallas import tpu as pltpu
from jax.experimental.pallas import tpu_sc as plsc

_N = 1_600_000
_G = 10_000
_STEP = 0.01
_NC = 2
_NS = 16
_NW = _NC * _NS
_NT = _N // _NW
_CH = 2_000
_NCHUNK = _NT // _CH
_L = 16
_GP = 10_240
_GC = _GP // _NS


def _hist_body(loss_hbm, ids_hbm, out_sums, out_counts,
               sums, counts, ids_buf, loss_buf, blk, red, shs, shc):
    cid = lax.axis_index("c")
    sid = lax.axis_index("s")
    wid = cid * _NS + sid

    zero = jnp.zeros((_L,), jnp.float32)
    ones = jnp.ones((_L,), jnp.float32)

    def zbody(i, _):
        sums[pl.ds(i * _L, _L)] = zero
        counts[pl.ds(i * _L, _L)] = zero
        return 0
    lax.fori_loop(0, _GP // _L, zbody, 0)

    base = wid * _NT

    def chunk_body(c, _):
        off = base + c * _CH
        pltpu.sync_copy(ids_hbm.at[pl.ds(off, _CH)], ids_buf)
        pltpu.sync_copy(loss_hbm.at[pl.ds(off, _CH)], loss_buf)

        def vbody(i, _):
            idx = ids_buf[pl.ds(i * _L, _L)]
            vals = loss_buf[pl.ds(i * _L, _L)]
            plsc.addupdate_scatter(sums, [idx], vals)
            plsc.addupdate_scatter(counts, [idx], ones)
            return 0
        lax.fori_loop(0, _CH // _L, vbody, 0)
        return 0
    lax.fori_loop(0, _NCHUNK, chunk_body, 0)

    pltpu.sync_copy(sums, shs.at[sid])
    pltpu.sync_copy(counts, shc.at[sid])
    plsc.subcore_barrier()

    gbase = sid * _GC
    for sh, out in ((shs, out_sums), (shc, out_counts)):
        pltpu.sync_copy(sh.at[:, pl.ds(gbase, _GC)], blk)

        def rbody(j, _):
            acc = blk[0, pl.ds(j * _L, _L)]
            for r in range(1, _NS):
                acc = acc + blk[r, pl.ds(j * _L, _L)]
            red[pl.ds(j * _L, _L)] = acc
            return 0
        lax.fori_loop(0, _GC // _L, rbody, 0)
        pltpu.sync_copy(red, out.at[cid, pl.ds(gbase, _GC)])


_PADL = 128


def _finish_body(ps_hbm, pc_hbm, w_hbm, out_hbm,
                 a0, a1, b0, b1, wv, st_buf, all_buf, outv, shared):
    cid = lax.axis_index("c")
    sid = lax.axis_index("s")

    @pl.when(cid == 0)
    def _():
        gbase = sid * _GC
        pltpu.sync_copy(ps_hbm.at[0, pl.ds(gbase, _GC)], a0)
        pltpu.sync_copy(ps_hbm.at[1, pl.ds(gbase, _GC)], a1)
        pltpu.sync_copy(pc_hbm.at[0, pl.ds(gbase, _GC)], b0)
        pltpu.sync_copy(pc_hbm.at[1, pl.ds(gbase, _GC)], b1)
        pltpu.sync_copy(w_hbm.at[pl.ds(gbase, _GC)], wv)

        zero = jnp.zeros((_L,), jnp.float32)

        def body(j, carry):
            s_acc, t_acc = carry
            d = pl.ds(j * _L, _L)
            s = a0[d] + a1[d]
            c = b0[d] + b1[d]
            gl = s / jnp.maximum(c, 1.0)
            u = wv[d] * jnp.exp(_STEP * gl)
            return (s_acc + u, t_acc + u * gl)

        s_acc, t_acc = lax.fori_loop(0, _GC // _L, body, (zero, zero))
        st_buf[0, 0:_L] = s_acc
        st_buf[1, 0:_L] = t_acc
        pltpu.sync_copy(st_buf, shared.at[sid])
        plsc.subcore_barrier()

        @pl.when(sid == 0)
        def _():
            pltpu.sync_copy(shared, all_buf)
            s_tot = all_buf[0, 0, 0:_L]
            t_tot = all_buf[0, 1, 0:_L]
            for r in range(1, _NS):
                s_tot = s_tot + all_buf[r, 0, 0:_L]
                t_tot = t_tot + all_buf[r, 1, 0:_L]
            s_b = jnp.full((_L,), jnp.sum(s_tot), jnp.float32)
            t_b = jnp.full((_L,), jnp.sum(t_tot), jnp.float32)
            outv[...] = t_b / s_b
            pltpu.sync_copy(outv, out_hbm)


def kernel(losses, group_ids, group_weights):
    mesh = plsc.VectorSubcoreMesh(core_axis_name="c", subcore_axis_name="s")

    hist = pl.kernel(
        _hist_body,
        out_type=(
            jax.ShapeDtypeStruct((_NC, _GP), jnp.float32),
            jax.ShapeDtypeStruct((_NC, _GP), jnp.float32),
        ),
        mesh=mesh,
        compiler_params=pltpu.CompilerParams(needs_layout_passes=False),
        scratch_types=[
            pltpu.VMEM((_GP,), jnp.float32),
            pltpu.VMEM((_GP,), jnp.float32),
            pltpu.VMEM((_CH,), jnp.int32),
            pltpu.VMEM((_CH,), jnp.float32),
            pltpu.VMEM((_NS, _GC), jnp.float32),
            pltpu.VMEM((_GC,), jnp.float32),
            pltpu.VMEM_SHARED((_NS, _GP), jnp.float32),
            pltpu.VMEM_SHARED((_NS, _GP), jnp.float32),
        ],
    )
    part_sums, part_counts = hist(losses, group_ids)

    w_pad = jnp.pad(group_weights, (0, _GP - _G))

    finish = pl.kernel(
        _finish_body,
        out_type=jax.ShapeDtypeStruct((_L,), jnp.float32),
        mesh=mesh,
        compiler_params=pltpu.CompilerParams(needs_layout_passes=False),
        scratch_types=[
            pltpu.VMEM((_GC,), jnp.float32),
            pltpu.VMEM((_GC,), jnp.float32),
            pltpu.VMEM((_GC,), jnp.float32),
            pltpu.VMEM((_GC,), jnp.float32),
            pltpu.VMEM((_GC,), jnp.float32),
            pltpu.VMEM((2, _PADL), jnp.float32),
            pltpu.VMEM((_NS, 2, _PADL), jnp.float32),
            pltpu.VMEM((_L,), jnp.float32),
            pltpu.VMEM_SHARED((_NS, 2, _PADL), jnp.float32),
        ],
    )
    out = finish(part_sums, part_counts, w_pad)
    return out[0]


# speedup vs baseline: 58.1267x; 1.5906x over previous
import functools

import jax
import jax.numpy as jnp
from jax import lax
from jax.experimental import pallas as pl
from jax.experimental.pallas import tpu as pltpu
from jax.experimental.pallas import tpu_sc as plsc

_N = 1_600_000
_G = 10_000
_STEP = 0.01
_NC = 2
_NS = 16
_NW = _NC * _NS
_NT = _N // _NW
_CH = 2_000
_NCHUNK = _NT // _CH
_L = 16
_UNROLL = 5
_GP = 10_240
_GC = _GP // _NS


def _hist_body(loss_hbm, ids_hbm, out_sums, out_counts,
               sums, counts, ids0, ids1, loss0, loss1, blk, red, shs, shc,
               sem0, sem1):
    cid = lax.axis_index("c")
    sid = lax.axis_index("s")
    wid = cid * _NS + sid

    zero = jnp.zeros((_L,), jnp.float32)
    ones = jnp.ones((_L,), jnp.float32)

    def zbody(i, _):
        for u in range(8):
            d = pl.ds((i * 8 + u) * _L, _L)
            sums[d] = zero
            counts[d] = zero
        return 0
    lax.fori_loop(0, _GP // (_L * 8), zbody, 0)

    base = wid * _NT
    bufs = ((ids0, loss0, sem0), (ids1, loss1, sem1))

    def start(c, b):
        ib, lb, sem = bufs[b]
        off = base + c * _CH
        pltpu.make_async_copy(ids_hbm.at[pl.ds(off, _CH)], ib, sem).start()
        pltpu.make_async_copy(loss_hbm.at[pl.ds(off, _CH)], lb, sem).start()

    def wait(b):
        ib, lb, sem = bufs[b]
        pltpu.make_async_copy(ids_hbm.at[pl.ds(0, _CH)], ib, sem).wait()
        pltpu.make_async_copy(loss_hbm.at[pl.ds(0, _CH)], lb, sem).wait()

    def process(b):
        ib, lb, _ = bufs[b]

        def vbody(i, _):
            for u in range(_UNROLL):
                d = pl.ds((i * _UNROLL + u) * _L, _L)
                idx = ib[d]
                vals = lb[d]
                plsc.addupdate_scatter(sums, [idx], vals)
                plsc.addupdate_scatter(counts, [idx], ones)
            return 0
        lax.fori_loop(0, _CH // (_L * _UNROLL), vbody, 0)

    start(0, 0)
    start(1, 1)

    def chunk_body(k, _):
        c = k * 2
        wait(0)
        process(0)
        start(c + 2, 0)
        wait(1)
        process(1)

        @pl.when(c + 3 < _NCHUNK)
        def _():
            start(c + 3, 1)
        return 0
    lax.fori_loop(0, _NCHUNK // 2, chunk_body, 0)
    wait(0)
    process(0)

    pltpu.sync_copy(sums, shs.at[sid])
    pltpu.sync_copy(counts, shc.at[sid])
    plsc.subcore_barrier()

    gbase = sid * _GC
    for sh, out in ((shs, out_sums), (shc, out_counts)):
        pltpu.sync_copy(sh.at[:, pl.ds(gbase, _GC)], blk)

        def rbody(j, _):
            acc = blk[0, pl.ds(j * _L, _L)]
            for r in range(1, _NS):
                acc = acc + blk[r, pl.ds(j * _L, _L)]
            red[pl.ds(j * _L, _L)] = acc
            return 0
        lax.fori_loop(0, _GC // _L, rbody, 0)
        pltpu.sync_copy(red, out.at[cid, pl.ds(gbase, _GC)])


_PADL = 128


def _finish_body(ps_hbm, pc_hbm, w_hbm, out_hbm,
                 a0, a1, b0, b1, wv, st_buf, all_buf, outv, shared):
    cid = lax.axis_index("c")
    sid = lax.axis_index("s")

    @pl.when(cid == 0)
    def _():
        gbase = sid * _GC
        pltpu.sync_copy(ps_hbm.at[0, pl.ds(gbase, _GC)], a0)
        pltpu.sync_copy(ps_hbm.at[1, pl.ds(gbase, _GC)], a1)
        pltpu.sync_copy(pc_hbm.at[0, pl.ds(gbase, _GC)], b0)
        pltpu.sync_copy(pc_hbm.at[1, pl.ds(gbase, _GC)], b1)
        pltpu.sync_copy(w_hbm.at[pl.ds(gbase, _GC)], wv)

        zero = jnp.zeros((_L,), jnp.float32)

        def body(j, carry):
            s_acc, t_acc = carry
            d = pl.ds(j * _L, _L)
            s = a0[d] + a1[d]
            c = b0[d] + b1[d]
            gl = s / jnp.maximum(c, 1.0)
            u = wv[d] * jnp.exp(_STEP * gl)
            return (s_acc + u, t_acc + u * gl)

        s_acc, t_acc = lax.fori_loop(0, _GC // _L, body, (zero, zero))
        st_buf[0, 0:_L] = s_acc
        st_buf[1, 0:_L] = t_acc
        pltpu.sync_copy(st_buf, shared.at[sid])
        plsc.subcore_barrier()

        @pl.when(sid == 0)
        def _():
            pltpu.sync_copy(shared, all_buf)
            s_tot = all_buf[0, 0, 0:_L]
            t_tot = all_buf[0, 1, 0:_L]
            for r in range(1, _NS):
                s_tot = s_tot + all_buf[r, 0, 0:_L]
                t_tot = t_tot + all_buf[r, 1, 0:_L]
            s_b = jnp.full((_L,), jnp.sum(s_tot), jnp.float32)
            t_b = jnp.full((_L,), jnp.sum(t_tot), jnp.float32)
            outv[...] = t_b / s_b
            pltpu.sync_copy(outv, out_hbm)


def kernel(losses, group_ids, group_weights):
    mesh = plsc.VectorSubcoreMesh(core_axis_name="c", subcore_axis_name="s")

    hist = pl.kernel(
        _hist_body,
        out_type=(
            jax.ShapeDtypeStruct((_NC, _GP), jnp.float32),
            jax.ShapeDtypeStruct((_NC, _GP), jnp.float32),
        ),
        mesh=mesh,
        compiler_params=pltpu.CompilerParams(needs_layout_passes=False),
        scratch_types=[
            pltpu.VMEM((_GP,), jnp.float32),
            pltpu.VMEM((_GP,), jnp.float32),
            pltpu.VMEM((_CH,), jnp.int32),
            pltpu.VMEM((_CH,), jnp.int32),
            pltpu.VMEM((_CH,), jnp.float32),
            pltpu.VMEM((_CH,), jnp.float32),
            pltpu.VMEM((_NS, _GC), jnp.float32),
            pltpu.VMEM((_GC,), jnp.float32),
            pltpu.VMEM_SHARED((_NS, _GP), jnp.float32),
            pltpu.VMEM_SHARED((_NS, _GP), jnp.float32),
            pltpu.SemaphoreType.DMA,
            pltpu.SemaphoreType.DMA,
        ],
    )
    part_sums, part_counts = hist(losses, group_ids)

    w_pad = jnp.pad(group_weights, (0, _GP - _G))

    finish = pl.kernel(
        _finish_body,
        out_type=jax.ShapeDtypeStruct((_L,), jnp.float32),
        mesh=mesh,
        compiler_params=pltpu.CompilerParams(needs_layout_passes=False),
        scratch_types=[
            pltpu.VMEM((_GC,), jnp.float32),
            pltpu.VMEM((_GC,), jnp.float32),
            pltpu.VMEM((_GC,), jnp.float32),
            pltpu.VMEM((_GC,), jnp.float32),
            pltpu.VMEM((_GC,), jnp.float32),
            pltpu.VMEM((2, _PADL), jnp.float32),
            pltpu.VMEM((_NS, 2, _PADL), jnp.float32),
            pltpu.VMEM((_L,), jnp.float32),
            pltpu.VMEM_SHARED((_NS, 2, _PADL), jnp.float32),
        ],
    )
    out = finish(part_sums, part_counts, w_pad)
    return out[0]


# speedup vs baseline: 62.2780x; 1.0714x over previous
import functools

import jax
import jax.numpy as jnp
from jax import lax
from jax.experimental import pallas as pl
from jax.experimental.pallas import tpu as pltpu
from jax.experimental.pallas import tpu_sc as plsc

_N = 1_600_000
_G = 10_000
_STEP = 0.01
_NC = 2
_NS = 16
_NW = _NC * _NS
_NT = _N // _NW
_CH = 2_000
_NCHUNK = _NT // _CH
_L = 16
_UNROLL = 25
_GP = 10_240
_GC = _GP // _NS


def _hist_body(loss_hbm, ids_hbm, out_sums, out_counts,
               sums, counts, ids0, ids1, loss0, loss1, blk, red, shs, shc,
               sem0, sem1):
    cid = lax.axis_index("c")
    sid = lax.axis_index("s")
    wid = cid * _NS + sid

    zero = jnp.zeros((_L,), jnp.float32)
    ones = jnp.ones((_L,), jnp.float32)

    def zbody(i, _):
        for u in range(8):
            d = pl.ds((i * 8 + u) * _L, _L)
            sums[d] = zero
            counts[d] = zero
        return 0
    lax.fori_loop(0, _GP // (_L * 8), zbody, 0)

    base = wid * _NT
    bufs = ((ids0, loss0, sem0), (ids1, loss1, sem1))

    def start(c, b):
        ib, lb, sem = bufs[b]
        off = base + c * _CH
        pltpu.make_async_copy(ids_hbm.at[pl.ds(off, _CH)], ib, sem).start()
        pltpu.make_async_copy(loss_hbm.at[pl.ds(off, _CH)], lb, sem).start()

    def wait(b):
        ib, lb, sem = bufs[b]
        pltpu.make_async_copy(ids_hbm.at[pl.ds(0, _CH)], ib, sem).wait()
        pltpu.make_async_copy(loss_hbm.at[pl.ds(0, _CH)], lb, sem).wait()

    def process(b):
        ib, lb, _ = bufs[b]

        def vbody(i, _):
            for u in range(_UNROLL):
                d = pl.ds((i * _UNROLL + u) * _L, _L)
                idx = ib[d]
                vals = lb[d]
                plsc.addupdate_scatter(sums, [idx], vals)
                plsc.addupdate_scatter(counts, [idx], ones)
            return 0
        lax.fori_loop(0, _CH // (_L * _UNROLL), vbody, 0)

    start(0, 0)
    start(1, 1)

    def chunk_body(k, _):
        c = k * 2
        wait(0)
        process(0)
        start(c + 2, 0)
        wait(1)
        process(1)

        @pl.when(c + 3 < _NCHUNK)
        def _():
            start(c + 3, 1)
        return 0
    lax.fori_loop(0, _NCHUNK // 2, chunk_body, 0)
    wait(0)
    process(0)

    pltpu.sync_copy(sums, shs.at[sid])
    pltpu.sync_copy(counts, shc.at[sid])
    plsc.subcore_barrier()

    gbase = sid * _GC
    for sh, out in ((shs, out_sums), (shc, out_counts)):
        pltpu.sync_copy(sh.at[:, pl.ds(gbase, _GC)], blk)

        def rbody(j, _):
            acc = blk[0, pl.ds(j * _L, _L)]
            for r in range(1, _NS):
                acc = acc + blk[r, pl.ds(j * _L, _L)]
            red[pl.ds(j * _L, _L)] = acc
            return 0
        lax.fori_loop(0, _GC // _L, rbody, 0)
        pltpu.sync_copy(red, out.at[cid, pl.ds(gbase, _GC)])


def _finish_tc(ps_ref, pc_ref, w_ref, out_ref):
    s = ps_ref[0] + ps_ref[1]
    c = pc_ref[0] + pc_ref[1]
    gl = s / jnp.maximum(c, 1.0)
    u = w_ref[...] * jnp.exp(_STEP * gl)
    s_tot = jnp.sum(u)
    t_tot = jnp.sum(u * gl)
    out_ref[...] = jnp.broadcast_to(t_tot / s_tot, (1, 1))


def kernel(losses, group_ids, group_weights):
    mesh = plsc.VectorSubcoreMesh(core_axis_name="c", subcore_axis_name="s")

    hist = pl.kernel(
        _hist_body,
        out_type=(
            jax.ShapeDtypeStruct((_NC, _GP), jnp.float32),
            jax.ShapeDtypeStruct((_NC, _GP), jnp.float32),
        ),
        mesh=mesh,
        compiler_params=pltpu.CompilerParams(needs_layout_passes=False),
        scratch_types=[
            pltpu.VMEM((_GP,), jnp.float32),
            pltpu.VMEM((_GP,), jnp.float32),
            pltpu.VMEM((_CH,), jnp.int32),
            pltpu.VMEM((_CH,), jnp.int32),
            pltpu.VMEM((_CH,), jnp.float32),
            pltpu.VMEM((_CH,), jnp.float32),
            pltpu.VMEM((_NS, _GC), jnp.float32),
            pltpu.VMEM((_GC,), jnp.float32),
            pltpu.VMEM_SHARED((_NS, _GP), jnp.float32),
            pltpu.VMEM_SHARED((_NS, _GP), jnp.float32),
            pltpu.SemaphoreType.DMA,
            pltpu.SemaphoreType.DMA,
        ],
    )
    part_sums, part_counts = hist(losses, group_ids)

    w_pad = jnp.pad(group_weights, (0, _GP - _G))

    ps3 = part_sums.reshape(_NC, _GP // 128, 128)
    pc3 = part_counts.reshape(_NC, _GP // 128, 128)
    w2 = w_pad.reshape(_GP // 128, 128)

    out = pl.pallas_call(
        _finish_tc,
        out_shape=jax.ShapeDtypeStruct((1, 1), jnp.float32),
    )(ps3, pc3, w2)
    return out[0, 0]
